# baseline (speedup 1.0000x reference)
import functools

import jax
import jax.numpy as jnp
from jax import lax
from jax.experimental import pallas as pl
from jax.experimental.pallas import tpu as pltpu
from jax.experimental.pallas import tpu_sc as plsc

NUM_CORES = 2
NUM_SUBCORES = 16
NW = NUM_CORES * NUM_SUBCORES
LANES = 16
B = 16384
D = 64
TW = 2 * D
BPW = B // NW
CHUNK = 128
NCHUNK = BPW // CHUNK
PASS_CHUNKS = 2
ROWS_PER_PASS = PASS_CHUNKS * CHUNK
NPASS = NCHUNK // PASS_CHUNKS
BLK_PER_PASS = ROWS_PER_PASS // LANES


def _mf_body(users_hbm, items_hbm, tab_u_hbm, tab_i_hbm, out_hbm,
             idx_u, idx_i, row_u, row_i, rows_u, rows_i, out_v, sem):
    wid = lax.axis_index("c") * NUM_SUBCORES + lax.axis_index("s")
    base = wid * BPW

    pltpu.sync_copy(users_hbm.at[wid], idx_u)
    pltpu.sync_copy(items_hbm.at[wid], idx_i)
    for j in range(NCHUNK):
        for k in range(CHUNK // LANES):
            s = pl.ds(k * LANES, LANES)
            row_u[j, s] = lax.shift_right_logical(idx_u[j, s], 1)
            row_i[j, s] = lax.shift_right_logical(idx_i[j, s], 1)

    for p in range(NPASS):
        copies = []
        for j in range(PASS_CHUNKS):
            c = p * PASS_CHUNKS + j
            dst = pl.ds(j * CHUNK, CHUNK)
            copies.append(pltpu.async_copy(
                tab_u_hbm.at[row_u.at[c]], rows_u.at[dst], sem))
            copies.append(pltpu.async_copy(
                tab_i_hbm.at[row_i.at[c]], rows_i.at[dst], sem))
        for cp in copies:
            cp.wait()

        for blk in range(BLK_PER_PASS):
            g = p * ROWS_PER_PASS + blk * LANES
            row_ids = blk * LANES + lax.iota(jnp.int32, LANES)
            iv = pl.ds(g, LANES)
            cb_u = lax.shift_left(idx_u[g // CHUNK, pl.ds(g % CHUNK, LANES)] & 1, 6)
            cb_i = lax.shift_left(idx_i[g // CHUNK, pl.ds(g % CHUNK, LANES)] & 1, 6)
            acc = jnp.zeros((LANES,), jnp.float32)
            for d in range(D):
                u = plsc.load_gather(rows_u, [row_ids, cb_u + d])
                v = plsc.load_gather(rows_i, [row_ids, cb_i + d])
                acc = acc + u * v
            out_v[iv] = 1.0 / (1.0 + jnp.exp(-acc))

    pltpu.sync_copy(out_v, out_hbm.at[pl.ds(base, BPW)])


@jax.jit
def _mf_call(users_r, items_r, tab_u, tab_i):
    mesh = plsc.VectorSubcoreMesh(core_axis_name="c", subcore_axis_name="s")
    run = functools.partial(
        pl.kernel,
        mesh=mesh,
        out_type=jax.ShapeDtypeStruct((B,), jnp.float32),
        scratch_types=[
            pltpu.VMEM((NCHUNK, CHUNK), jnp.int32),
            pltpu.VMEM((NCHUNK, CHUNK), jnp.int32),
            pltpu.VMEM((NCHUNK, CHUNK), jnp.int32),
            pltpu.VMEM((NCHUNK, CHUNK), jnp.int32),
            pltpu.VMEM((ROWS_PER_PASS, TW), jnp.float32),
            pltpu.VMEM((ROWS_PER_PASS, TW), jnp.float32),
            pltpu.VMEM((BPW,), jnp.float32),
            pltpu.SemaphoreType.DMA,
        ],
        compiler_params=pltpu.CompilerParams(needs_layout_passes=False),
    )(_mf_body)
    return run(users_r, items_r, tab_u, tab_i)


def kernel(users, items, embedding_user, embedding_item):
    users_r = users.reshape(NW, NCHUNK, CHUNK)
    items_r = items.reshape(NW, NCHUNK, CHUNK)
    tab_u = embedding_user.reshape(1000000 * D // TW, TW)
    tab_i = embedding_item.reshape(1000000 * D // TW, TW)
    return _mf_call(users_r, items_r, tab_u, tab_i)
